# baseline (speedup 1.0000x reference)
import functools

import jax
import jax.numpy as jnp
from jax import lax
from jax.experimental import pallas as pl
from jax.experimental.pallas import tpu as pltpu
from jax.experimental.pallas import tpu_sc as plsc

_L = 16


def _take(vec, idx):
    return jnp.take_along_axis(vec, idx, axis=0, mode="promise_in_bounds")


def _make_sc_gather(n_rows, n_cols, n_workers, chunk_rows, d):
    per_worker = n_rows // n_workers
    n_chunks = per_worker // chunk_rows
    cvals = chunk_rows * n_cols
    mesh = plsc.VectorSubcoreMesh(core_axis_name="c", subcore_axis_name="s")

    @functools.partial(
        pl.kernel,
        mesh=mesh,
        out_type=jax.ShapeDtypeStruct((n_rows, n_cols, d), jnp.float32),
        scratch_types=[
            pltpu.VMEM((8, d), jnp.float32),
            pltpu.VMEM((chunk_rows, n_cols), jnp.int32),
            pltpu.VMEM((chunk_rows, n_cols), jnp.int32),
            pltpu.VMEM((chunk_rows, n_cols, d), jnp.float32),
            pltpu.VMEM((chunk_rows, n_cols, d), jnp.float32),
            pltpu.SemaphoreType.DMA,
            pltpu.SemaphoreType.DMA,
            pltpu.SemaphoreType.DMA,
            pltpu.SemaphoreType.DMA,
        ],
        compiler_params=pltpu.CompilerParams(
            needs_layout_passes=False, use_tc_tiling_on_sc=False
        ),
    )
    def k(tab_hbm, x_hbm, out_hbm, tab_v, ib0, ib1, ob0, ob1, si0, si1, so0, so1):
        nc = 2
        wid = lax.axis_index("s") * nc + lax.axis_index("c")
        row0 = wid * per_worker
        pltpu.sync_copy(tab_hbm, tab_v)
        lane = lax.iota(jnp.int32, _L)
        rep_pat = lane // 4
        col_pat = lane & 3
        tf_lo = plsc.load_gather(tab_v, [rep_pat, col_pat])
        tf_hi = plsc.load_gather(tab_v, [rep_pat + 4, col_pat])
        sem_in = [si0, si1]
        sem_out = [so0, so1]
        idx_bufs = [ib0, ib1]
        out_bufs = [ob0, ob1]

        def start_in(c):
            return pltpu.async_copy(
                x_hbm.at[pl.ds(row0 + c * chunk_rows, chunk_rows)],
                idx_bufs[c % 2],
                sem_in[c % 2],
            )

        def start_out(c):
            return pltpu.async_copy(
                out_bufs[c % 2],
                out_hbm.at[pl.ds(row0 + c * chunk_rows, chunk_rows)],
                sem_out[c % 2],
            )

        cp_in = {0: start_in(0)}
        cp_out = {}
        for c in range(n_chunks):
            b = c % 2
            if c + 1 < n_chunks:
                cp_in[c + 1] = start_in(c + 1)
            cp_in[c].wait()
            if c >= 2:
                cp_out[c - 2].wait()
            idx_c = idx_bufs[b]
            out_c = out_bufs[b]

            @plsc.parallel_loop(0, cvals, _L, unroll=4)
            def body(base):
                f = base + lane
                fi = f // n_cols
                fs = f - fi * n_cols
                xi = plsc.load_gather(idx_c, [fi, fs])
                for k_ in range(4):
                    t = base + 4 * k_ + rep_pat
                    rep = _take(xi, rep_pat + 4 * k_)
                    gidx = rep * d + col_pat
                    g15 = gidx & (_L - 1)
                    v = jnp.where(gidx < _L, _take(tf_lo, g15), _take(tf_hi, g15))
                    ti = t // n_cols
                    ts = t - ti * n_cols
                    plsc.store_scatter(out_c, [ti, ts, col_pat], v)

            cp_out[c] = start_out(c)
        cp_out[n_chunks - 2].wait()
        cp_out[n_chunks - 1].wait()

    return k


def kernel(x, table):
    b, s = x.shape
    v, d = table.shape
    out = _make_sc_gather(b, s, 32, 64, d)(table, x.astype(jnp.int32))
    return out


# speedup vs baseline: 10.8919x; 10.8919x over previous
import functools

import jax
import jax.numpy as jnp
from jax import lax
from jax.experimental import pallas as pl
from jax.experimental.pallas import tpu as pltpu
from jax.experimental.pallas import tpu_sc as plsc

_L = 16


def _take(vec, idx):
    return jnp.take_along_axis(vec, idx, axis=0, mode="promise_in_bounds")


def _make_sc_gather(n_rows, n_cols, n_workers, chunk_cols, d):
    per_worker = n_rows // n_workers
    n_chunks = per_worker // chunk_cols
    nvec = n_cols * chunk_cols // _L
    rv_per_s = chunk_cols // _L
    mesh = plsc.VectorSubcoreMesh(core_axis_name="c", subcore_axis_name="s")

    @functools.partial(
        pl.kernel,
        mesh=mesh,
        out_type=jax.ShapeDtypeStruct((n_cols, d, n_rows), jnp.float32),
        scratch_types=[
            pltpu.VMEM((8, d), jnp.float32),
            pltpu.VMEM((n_cols, chunk_cols), jnp.int32),
            pltpu.VMEM((n_cols, chunk_cols), jnp.int32),
            pltpu.VMEM((n_cols, d, chunk_cols), jnp.float32),
            pltpu.VMEM((n_cols, d, chunk_cols), jnp.float32),
            pltpu.SemaphoreType.DMA,
            pltpu.SemaphoreType.DMA,
            pltpu.SemaphoreType.DMA,
            pltpu.SemaphoreType.DMA,
        ],
        compiler_params=pltpu.CompilerParams(
            needs_layout_passes=False, use_tc_tiling_on_sc=False
        ),
    )
    def k(tab_hbm, xt_hbm, out_hbm, tab_v, ib0, ib1, ob0, ob1, si0, si1, so0, so1):
        nc = 2
        wid = lax.axis_index("s") * nc + lax.axis_index("c")
        col0 = wid * per_worker
        pltpu.sync_copy(tab_hbm, tab_v)
        lane = lax.iota(jnp.int32, _L)
        row_pat = lane & 7
        tcols = [
            plsc.load_gather(tab_v, [row_pat, jnp.full((_L,), j, jnp.int32)])
            for j in range(d)
        ]
        sem_in = [si0, si1]
        sem_out = [so0, so1]
        idx_bufs = [ib0, ib1]
        out_bufs = [ob0, ob1]

        def start_in(c):
            return pltpu.async_copy(
                xt_hbm.at[:, pl.ds(col0 + c * chunk_cols, chunk_cols)],
                idx_bufs[c % 2],
                sem_in[c % 2],
            )

        def start_out(c):
            return pltpu.async_copy(
                out_bufs[c % 2],
                out_hbm.at[:, :, pl.ds(col0 + c * chunk_cols, chunk_cols)],
                sem_out[c % 2],
            )

        cp_in = {0: start_in(0)}
        cp_out = {}
        for c in range(n_chunks):
            b = c % 2
            if c + 1 < n_chunks:
                cp_in[c + 1] = start_in(c + 1)
            cp_in[c].wait()
            if c >= 2:
                cp_out[c - 2].wait()
            idx_c = idx_bufs[b]
            out_c = out_bufs[b]

            @plsc.parallel_loop(0, nvec, 1, unroll=4)
            def body(q):
                s = q // rv_per_s
                rv = (q - s * rv_per_s) * _L
                xi = idx_c[s, pl.ds(rv, _L)]
                for j in range(d):
                    out_c[s, j, pl.ds(rv, _L)] = _take(tcols[j], xi)

            cp_out[c] = start_out(c)
        cp_out[n_chunks - 2].wait()
        cp_out[n_chunks - 1].wait()

    return k


def kernel(x, table):
    b, s = x.shape
    v, d = table.shape
    xt = jnp.transpose(x).astype(jnp.int32)
    out_t = _make_sc_gather(b, s, 32, 128, d)(table, xt)
    return jnp.transpose(out_t, (2, 0, 1))


# speedup vs baseline: 19.1554x; 1.7587x over previous
import functools

import jax
import jax.numpy as jnp
from jax import lax
from jax.experimental import pallas as pl
from jax.experimental.pallas import tpu as pltpu
from jax.experimental.pallas import tpu_sc as plsc

_L = 16


def _take(vec, idx):
    return jnp.take_along_axis(vec, idx, axis=0, mode="promise_in_bounds")


def _make_sc_gather(n_rows, n_cols, n_workers, chunk_cols, d):
    per_worker = n_rows // n_workers
    n_chunks = per_worker // chunk_cols
    nvec = n_cols * chunk_cols // _L
    rv_per_s = chunk_cols // _L
    mesh = plsc.VectorSubcoreMesh(core_axis_name="c", subcore_axis_name="s")

    @functools.partial(
        pl.kernel,
        mesh=mesh,
        out_type=jax.ShapeDtypeStruct(
            (n_cols, n_rows // chunk_cols, d, chunk_cols), jnp.float32
        ),
        scratch_types=[
            pltpu.VMEM((8, d), jnp.float32),
            pltpu.VMEM((n_cols, chunk_cols), jnp.int32),
            pltpu.VMEM((n_cols, chunk_cols), jnp.int32),
            pltpu.VMEM((n_cols, d, chunk_cols), jnp.float32),
            pltpu.VMEM((n_cols, d, chunk_cols), jnp.float32),
            pltpu.SemaphoreType.DMA,
            pltpu.SemaphoreType.DMA,
            pltpu.SemaphoreType.DMA,
            pltpu.SemaphoreType.DMA,
        ],
        compiler_params=pltpu.CompilerParams(
            needs_layout_passes=False, use_tc_tiling_on_sc=False
        ),
    )
    def k(tab_hbm, xt_hbm, out_hbm, tab_v, ib0, ib1, ob0, ob1, si0, si1, so0, so1):
        nc = 2
        wid = lax.axis_index("s") * nc + lax.axis_index("c")
        col0 = wid * per_worker
        pltpu.sync_copy(tab_hbm, tab_v)
        lane = lax.iota(jnp.int32, _L)
        row_pat = lane & 7
        tcols = [
            plsc.load_gather(tab_v, [row_pat, jnp.full((_L,), j, jnp.int32)])
            for j in range(d)
        ]
        sem_in = [si0, si1]
        sem_out = [so0, so1]
        idx_bufs = [ib0, ib1]
        out_bufs = [ob0, ob1]

        def start_in(c):
            return pltpu.async_copy(
                xt_hbm.at[:, pl.ds(col0 + c * chunk_cols, chunk_cols)],
                idx_bufs[c % 2],
                sem_in[c % 2],
            )

        def start_out(c):
            return pltpu.async_copy(
                out_bufs[c % 2],
                out_hbm.at[:, wid * n_chunks + c],
                sem_out[c % 2],
            )

        cp_in = {0: start_in(0)}
        cp_out = {}
        for c in range(n_chunks):
            b = c % 2
            if c + 1 < n_chunks:
                cp_in[c + 1] = start_in(c + 1)
            cp_in[c].wait()
            if c >= 2:
                cp_out[c - 2].wait()
            idx_c = idx_bufs[b]
            out_c = out_bufs[b]

            @plsc.parallel_loop(0, nvec, 1, unroll=4)
            def body(q):
                s = q // rv_per_s
                rv = (q - s * rv_per_s) * _L
                xi = idx_c[s, pl.ds(rv, _L)]
                for j in range(d):
                    out_c[s, j, pl.ds(rv, _L)] = _take(tcols[j], xi)

            cp_out[c] = start_out(c)
        cp_out[n_chunks - 2].wait()
        cp_out[n_chunks - 1].wait()

    return k


def kernel(x, table):
    b, s = x.shape
    v, d = table.shape
    xt = jnp.transpose(x).astype(jnp.int32)
    out4 = _make_sc_gather(b, s, 32, 128, d)(table, xt)
    return jnp.transpose(out4, (1, 3, 0, 2)).reshape(b, s, d)


# speedup vs baseline: 19.1704x; 1.0008x over previous
import functools

import jax
import jax.numpy as jnp
from jax import lax
from jax.experimental import pallas as pl
from jax.experimental.pallas import tpu as pltpu
from jax.experimental.pallas import tpu_sc as plsc

_L = 16


def _take(vec, idx):
    return jnp.take_along_axis(vec, idx, axis=0, mode="promise_in_bounds")


def _make_sc_gather(n_rows, n_cols, n_workers, chunk_cols, d, blk_cols=128):
    per_worker = n_rows // n_workers
    n_chunks = per_worker // chunk_cols
    nvec = n_cols * chunk_cols // _L
    rv_per_s = chunk_cols // _L
    per_blk = blk_cols // chunk_cols
    mesh = plsc.VectorSubcoreMesh(core_axis_name="c", subcore_axis_name="s")

    @functools.partial(
        pl.kernel,
        mesh=mesh,
        out_type=jax.ShapeDtypeStruct(
            (n_cols, n_rows // blk_cols, d, blk_cols), jnp.float32
        ),
        scratch_types=[
            pltpu.VMEM((8, d), jnp.float32),
            pltpu.VMEM((n_cols, chunk_cols), jnp.int32),
            pltpu.VMEM((n_cols, chunk_cols), jnp.int32),
            pltpu.VMEM((n_cols, d, chunk_cols), jnp.float32),
            pltpu.VMEM((n_cols, d, chunk_cols), jnp.float32),
            pltpu.SemaphoreType.DMA,
            pltpu.SemaphoreType.DMA,
            pltpu.SemaphoreType.DMA,
            pltpu.SemaphoreType.DMA,
        ],
        compiler_params=pltpu.CompilerParams(
            needs_layout_passes=False, use_tc_tiling_on_sc=False
        ),
    )
    def k(tab_hbm, xt_hbm, out_hbm, tab_v, ib0, ib1, ob0, ob1, si0, si1, so0, so1):
        nc = 2
        wid = lax.axis_index("s") * nc + lax.axis_index("c")
        col0 = wid * per_worker
        pltpu.sync_copy(tab_hbm, tab_v)
        lane = lax.iota(jnp.int32, _L)
        row_pat = lane & 7
        tcols = [
            plsc.load_gather(tab_v, [row_pat, jnp.full((_L,), j, jnp.int32)])
            for j in range(d)
        ]
        sem_in = [si0, si1]
        sem_out = [so0, so1]
        idx_bufs = [ib0, ib1]
        out_bufs = [ob0, ob1]

        def start_in(c):
            return pltpu.async_copy(
                xt_hbm.at[:, pl.ds(col0 + c * chunk_cols, chunk_cols)],
                idx_bufs[c % 2],
                sem_in[c % 2],
            )

        def start_out(c):
            blk = wid * (per_worker // blk_cols) + c // per_blk
            off = (c % per_blk) * chunk_cols
            return pltpu.async_copy(
                out_bufs[c % 2],
                out_hbm.at[:, blk, :, pl.ds(off, chunk_cols)],
                sem_out[c % 2],
            )

        cp_in = {0: start_in(0)}
        cp_out = {}
        for c in range(n_chunks):
            b = c % 2
            if c + 1 < n_chunks:
                cp_in[c + 1] = start_in(c + 1)
            cp_in[c].wait()
            if c >= 2:
                cp_out[c - 2].wait()
            idx_c = idx_bufs[b]
            out_c = out_bufs[b]

            @plsc.parallel_loop(0, nvec, 1, unroll=4)
            def body(q):
                s = q // rv_per_s
                rv = (q - s * rv_per_s) * _L
                xi = idx_c[s, pl.ds(rv, _L)]
                for j in range(d):
                    out_c[s, j, pl.ds(rv, _L)] = _take(tcols[j], xi)

            cp_out[c] = start_out(c)
        cp_out[n_chunks - 2].wait()
        cp_out[n_chunks - 1].wait()

    return k


def kernel(x, table):
    b, s = x.shape
    v, d = table.shape
    xt = jnp.transpose(x).astype(jnp.int32)
    out4 = _make_sc_gather(b, s, 32, 128, d)(table, xt)
    return jnp.transpose(out4, (1, 3, 0, 2)).reshape(b, s, d)
